# baseline (speedup 1.0000x reference)
import functools

import jax
import jax.numpy as jnp
from jax import lax
from jax.experimental import pallas as pl
from jax.experimental.pallas import tpu as pltpu
from jax.experimental.pallas import tpu_sc as plsc

BATCH = 8
SEQ = 1024
RED = 1024
DMODEL = 512
NROWS = BATCH * SEQ

NUM_WORKERS = 32
ROWS_PER_WORKER = NROWS // NUM_WORKERS
CHUNK = 64


def _counts_body(d_ref, idx_ref):
    d = d_ref[...]
    hit = jnp.logical_or(d == 1, d == -1)
    c = jnp.sum(hit.astype(jnp.int32), axis=-1)
    idx_ref[...] = jnp.minimum(c, DMODEL - 1).reshape(1, 1, SEQ)


def _centrality_counts(distances):
    return pl.pallas_call(
        _counts_body,
        grid=(BATCH,),
        in_specs=[pl.BlockSpec((1, SEQ, RED), lambda b: (b, 0, 0))],
        out_specs=pl.BlockSpec((1, 1, SEQ), lambda b: (b, 0, 0)),
        out_shape=jax.ShapeDtypeStruct((BATCH, 1, SEQ), jnp.int32),
    )(distances)


def _gather_body(table_hbm, idx_hbm, out_hbm, idx_v, rows0, rows1,
                 gsem0, gsem1, osem0, osem1):
    sid = lax.axis_index("s")
    wid = sid * 2 + lax.axis_index("c")
    base = wid * ROWS_PER_WORKER

    pltpu.sync_copy(idx_hbm.at[pl.ds(base, ROWS_PER_WORKER)], idx_v)

    rows = (rows0, rows1)
    gsems = (gsem0, gsem1)
    osems = (osem0, osem1)
    nchunk = ROWS_PER_WORKER // CHUNK
    gathers = [None] * nchunk
    outs = [None] * nchunk
    for c in range(nchunk):
        b = c % 2
        if c >= 2:
            outs[c - 2].wait()
        gathers[c] = pltpu.async_copy(
            table_hbm.at[idx_v.at[pl.ds(c * CHUNK, CHUNK)]], rows[b], gsems[b]
        )
        gathers[c].wait()
        outs[c] = pltpu.async_copy(
            rows[b], out_hbm.at[pl.ds(base + c * CHUNK, CHUNK)], osems[b]
        )
    outs[nchunk - 2].wait()
    outs[nchunk - 1].wait()


@functools.lru_cache(maxsize=1)
def _gather_rows():
    return pl.kernel(
        _gather_body,
        mesh=plsc.VectorSubcoreMesh(core_axis_name="c", subcore_axis_name="s"),
        out_type=jax.ShapeDtypeStruct((NROWS, DMODEL), jnp.float32),
        scratch_types=[
            pltpu.VMEM((ROWS_PER_WORKER,), jnp.int32),
            pltpu.VMEM((CHUNK, DMODEL), jnp.float32),
            pltpu.VMEM((CHUNK, DMODEL), jnp.float32),
            pltpu.SemaphoreType.DMA,
            pltpu.SemaphoreType.DMA,
            pltpu.SemaphoreType.DMA,
            pltpu.SemaphoreType.DMA,
        ],
    )


def kernel(distances, table):
    idx = _centrality_counts(distances).reshape(NROWS)
    rows = _gather_rows()(table, idx)
    return rows.reshape(BATCH, SEQ, DMODEL)


# speedup vs baseline: 2.0482x; 2.0482x over previous
import functools

import jax
import jax.numpy as jnp
from jax import lax
from jax.experimental import pallas as pl
from jax.experimental.pallas import tpu as pltpu
from jax.experimental.pallas import tpu_sc as plsc

BATCH = 8
SEQ = 1024
RED = 1024
DMODEL = 512
NROWS = BATCH * SEQ

NUM_WORKERS = 32
ROWS_PER_WORKER = NROWS // NUM_WORKERS
CHUNK = 64


NREP = 8


def _counts_body(d_ref, t_ref, idx_ref, rep_ref):
    b = pl.program_id(0)
    d = d_ref[...]
    hit = jnp.logical_or(d == 1, d == -1)
    c = jnp.sum(hit.astype(jnp.int32), axis=-1)
    rep = (b * (SEQ // 256) + lax.broadcasted_iota(jnp.int32, (1, SEQ), 1) // 256) % NREP
    idx_ref[...] = (jnp.minimum(c, DMODEL - 1) + DMODEL * rep).reshape(1, 1, SEQ)
    rep_ref[...] = t_ref[...]


def _centrality_counts(distances, table):
    return pl.pallas_call(
        _counts_body,
        grid=(BATCH,),
        in_specs=[
            pl.BlockSpec((1, SEQ, RED), lambda b: (b, 0, 0)),
            pl.BlockSpec((DMODEL, DMODEL), lambda b: (0, 0)),
        ],
        out_specs=[
            pl.BlockSpec((1, 1, SEQ), lambda b: (b, 0, 0)),
            pl.BlockSpec((DMODEL, DMODEL), lambda b: (b % NREP, 0)),
        ],
        out_shape=[
            jax.ShapeDtypeStruct((BATCH, 1, SEQ), jnp.int32),
            jax.ShapeDtypeStruct((NREP * DMODEL, DMODEL), jnp.float32),
        ],
    )(distances, table)


def _gather_body(table_hbm, idx_hbm, out_hbm, idx_v, rows0, rows1,
                 gsem0, gsem1, osem0, osem1):
    sid = lax.axis_index("s")
    wid = sid * 2 + lax.axis_index("c")
    base = wid * ROWS_PER_WORKER

    pltpu.sync_copy(idx_hbm.at[pl.ds(base, ROWS_PER_WORKER)], idx_v)

    rows = (rows0, rows1)
    gsems = (gsem0, gsem1)
    osems = (osem0, osem1)
    nchunk = ROWS_PER_WORKER // CHUNK
    gathers = [None] * nchunk
    outs = [None] * nchunk
    for c in range(nchunk):
        b = c % 2
        if c >= 2:
            outs[c - 2].wait()
        gathers[c] = pltpu.async_copy(
            table_hbm.at[idx_v.at[pl.ds(c * CHUNK, CHUNK)]], rows[b], gsems[b]
        )
        gathers[c].wait()
        outs[c] = pltpu.async_copy(
            rows[b], out_hbm.at[pl.ds(base + c * CHUNK, CHUNK)], osems[b]
        )
    outs[nchunk - 2].wait()
    outs[nchunk - 1].wait()


@functools.lru_cache(maxsize=1)
def _gather_rows():
    return pl.kernel(
        _gather_body,
        mesh=plsc.VectorSubcoreMesh(core_axis_name="c", subcore_axis_name="s"),
        out_type=jax.ShapeDtypeStruct((NROWS, DMODEL), jnp.float32),
        scratch_types=[
            pltpu.VMEM((ROWS_PER_WORKER,), jnp.int32),
            pltpu.VMEM((CHUNK, DMODEL), jnp.float32),
            pltpu.VMEM((CHUNK, DMODEL), jnp.float32),
            pltpu.SemaphoreType.DMA,
            pltpu.SemaphoreType.DMA,
            pltpu.SemaphoreType.DMA,
            pltpu.SemaphoreType.DMA,
        ],
    )


def kernel(distances, table):
    idx, rep = _centrality_counts(distances, table)
    rows = _gather_rows()(rep, idx.reshape(NROWS))
    return rows.reshape(BATCH, SEQ, DMODEL)
